# Initial kernel scaffold; baseline (speedup 1.0000x reference)
import jax
import jax.numpy as jnp
from jax.experimental import pallas as pl


def kernel(v0, edge_index, W1, b1, W2, b2):
    raise NotImplementedError("write your pallas kernel here")


# speedup vs baseline: 4.4527x; 4.4527x over previous
import functools

import jax
import jax.numpy as jnp
from jax import lax
from jax.experimental import pallas as pl
from jax.experimental.pallas import tpu as pltpu
from jax.experimental.pallas import tpu_sc as plsc

_N = 10000
_D = 128
_E = 320000
_NP = 10240
_NC = 2
_NS = 16
_NW = _NC * _NS
_CH = 128
_NCHUNK = 79
_EPW = _NCHUNK * _CH
_RPT = _NP // _NS
_ZB = 128


def _mm_body(x_ref, w_ref, o_ref):
    o_ref[...] = jnp.dot(x_ref[...], w_ref[...],
                         preferred_element_type=jnp.float32)


def _matmul(x, w):
    bm = 1280
    return pl.pallas_call(
        _mm_body,
        grid=(_NP // bm,),
        in_specs=[pl.BlockSpec((bm, _D), lambda i: (i, 0)),
                  pl.BlockSpec((_D, _D), lambda i: (0, 0))],
        out_specs=pl.BlockSpec((bm, _D), lambda i: (i, 0)),
        out_shape=jax.ShapeDtypeStruct((_NP, _D), jnp.float32),
    )(x, w)


def _fuse_body(p_ref, b_ref, w_ref, o_ref):
    i = pl.program_id(0)
    bm = o_ref.shape[0]
    row = i * bm + lax.broadcasted_iota(jnp.int32, (bm, 1), 0)
    x = jnp.maximum(p_ref[0] + p_ref[1] + b_ref[...], 0.0)
    x = jnp.where(row < _N, x, 0.0)
    o_ref[...] = jnp.dot(x, w_ref[...], preferred_element_type=jnp.float32)


def _fused_relu_mm(p, b, w):
    bm = 1280
    return pl.pallas_call(
        _fuse_body,
        grid=(_NP // bm,),
        in_specs=[pl.BlockSpec((2, bm, _D), lambda i: (0, i, 0)),
                  pl.BlockSpec((1, _D), lambda i: (0, 0)),
                  pl.BlockSpec((_D, _D), lambda i: (0, 0))],
        out_specs=pl.BlockSpec((bm, _D), lambda i: (i, 0)),
        out_shape=jax.ShapeDtypeStruct((_NP, _D), jnp.float32),
    )(p, b, w)


def _final_body(q_ref, b_ref, o_ref):
    o_ref[...] = q_ref[0] + q_ref[1] + b_ref[...]


def _final_add(q, b):
    bm = 1280
    return pl.pallas_call(
        _final_body,
        grid=(_NP // bm,),
        in_specs=[pl.BlockSpec((2, bm, _D), lambda i: (0, i, 0)),
                  pl.BlockSpec((1, _D), lambda i: (0, 0))],
        out_specs=pl.BlockSpec((bm, _D), lambda i: (i, 0)),
        out_shape=jax.ShapeDtypeStruct((_NP, _D), jnp.float32),
    )(q, b)


def _seg_body(h_hbm, src_hbm, dst_hbm, out_hbm,
              src_v, dst_v, rows_v, acc, gsem):
    c = lax.axis_index("c")
    s = lax.axis_index("s")
    wid = c * _NS + s

    z = jnp.zeros((16,), jnp.float32)

    def zrow(i, carry):
        for j in range(_D // 16):
            rows_v[i, pl.ds(j * 16, 16)] = z
        return carry

    lax.fori_loop(0, _ZB, zrow, 0)
    for k in range(_RPT // _ZB):
        pltpu.sync_copy(rows_v, acc.at[pl.ds(s * _RPT + k * _ZB, _ZB)])
    plsc.subcore_barrier()

    pltpu.sync_copy(src_hbm.at[wid], src_v)
    pltpu.sync_copy(dst_hbm.at[wid], dst_v)

    def chunk(i, carry):
        pltpu.async_copy(h_hbm.at[src_v.at[i]], rows_v, gsem).wait()
        pltpu.sync_copy(rows_v, acc.at[dst_v.at[i]], add=True)
        return carry

    lax.fori_loop(0, _NCHUNK, chunk, 0)
    plsc.subcore_barrier()

    pltpu.sync_copy(acc.at[pl.ds(s * _RPT, _RPT)],
                    out_hbm.at[c, pl.ds(s * _RPT, _RPT)])


@functools.partial(
    pl.kernel,
    out_type=jax.ShapeDtypeStruct((_NC, _NP, _D), jnp.float32),
    mesh=plsc.VectorSubcoreMesh(core_axis_name="c", subcore_axis_name="s"),
    scratch_types=[
        pltpu.VMEM((_NCHUNK, _CH), jnp.int32),
        pltpu.VMEM((_NCHUNK, _CH), jnp.int32),
        pltpu.VMEM((_CH, _D), jnp.float32),
        pltpu.VMEM_SHARED((_NP, _D), jnp.float32),
        pltpu.SemaphoreType.DMA,
    ],
)
def _seg_partial(h_hbm, src_hbm, dst_hbm, out_hbm,
                 src_v, dst_v, rows_v, acc, gsem):
    _seg_body(h_hbm, src_hbm, dst_hbm, out_hbm,
              src_v, dst_v, rows_v, acc, gsem)


def kernel(v0, edge_index, W1, b1, W2, b2):
    npad = _NW * _EPW - _E
    src = jnp.concatenate(
        [edge_index[0].astype(jnp.int32),
         jnp.full((npad,), _N, jnp.int32)]).reshape(_NW, _NCHUNK, _CH)
    dst = jnp.concatenate(
        [edge_index[1].astype(jnp.int32),
         jnp.full((npad,), _N, jnp.int32)]).reshape(_NW, _NCHUNK, _CH)
    v0p = jnp.pad(v0.astype(jnp.float32), ((0, _NP - _N), (0, 0)))
    b1r = b1.reshape(1, _D).astype(jnp.float32)
    b2r = b2.reshape(1, _D).astype(jnp.float32)

    h1 = _matmul(v0p, W1)
    p = _seg_partial(h1, src, dst)
    h2 = _fused_relu_mm(p, b1r, W2)
    q = _seg_partial(h2, src, dst)
    return _final_add(q, b2r)[:_N]
